# baseline (speedup 1.0000x reference)
import functools

import jax
import jax.numpy as jnp
from jax import lax
from jax.experimental import pallas as pl
from jax.experimental.pallas import tpu as pltpu
from jax.experimental.pallas import tpu_sc as plsc

N = 10000
D = 128
K = 48
NP = 10240
ROWS_PER_W = 320
NUM_VREGS = 625
GRP = 5
NUM_GRPS = 125
BLK = 16


RBLK = 80


def _rank_body(A_ref, r_ref, deg_ref):
    a = A_ref[...] != 0.0
    inc = jnp.where(a, 1, 0)
    s = 1
    while s < N:
        sh = jnp.concatenate(
            [jnp.zeros((RBLK, s), jnp.int32), inc[:, :N - s]], axis=1)
        inc = inc + sh
        s *= 2
    r_ref[...] = jnp.where(a, inc, 0)
    deg_ref[...] = inc[:, N - 1:N]


def _tc_rank(A):
    return pl.pallas_call(
        _rank_body,
        grid=(N // RBLK,),
        in_specs=[pl.BlockSpec((RBLK, N), lambda i: (i, 0))],
        out_specs=[
            pl.BlockSpec((RBLK, N), lambda i: (i, 0)),
            pl.BlockSpec((RBLK, 1), lambda i: (i, 0)),
        ],
        out_shape=[
            jax.ShapeDtypeStruct((N, N), jnp.int32),
            jax.ShapeDtypeStruct((N, 1), jnp.int32),
        ],
    )(A)


def _sc_extract_gather(r32, x):
    info = plsc.get_sparse_core_info()
    nc = info.num_cores
    mesh = plsc.VectorSubcoreMesh(core_axis_name="c", subcore_axis_name="s")

    @functools.partial(
        pl.kernel,
        mesh=mesh,
        compiler_params=pltpu.CompilerParams(needs_layout_passes=False),
        out_type=jax.ShapeDtypeStruct((NP, K, D), jnp.float32),
        scratch_types=[
            pltpu.VMEM((2, N), jnp.int32),
            pltpu.VMEM((4, 64), jnp.int32),
            pltpu.VMEM((4, 64, D), jnp.float32),
            pltpu.SemaphoreType.DMA((2,)),
            pltpu.SemaphoreType.DMA,
            pltpu.SemaphoreType.DMA,
        ],
    )
    def sc_kernel(r_hbm, x_hbm, nf_out, row_v, nbr_v, rows_v, sem_r,
                  sem_g, sem_w):
        wid = lax.axis_index("s") * nc + lax.axis_index("c")
        base = wid * ROWS_PER_W
        nrows = jnp.maximum(jnp.minimum(ROWS_PER_W, N - base), 0)
        lanes = lax.iota(jnp.int32, 16)
        zeros16 = jnp.zeros((16,), jnp.int32)
        for q in range(4):
            for t in range(4):
                nbr_v[q, pl.ds(t * 16, 16)] = zeros16

        @pl.when(nrows > 0)
        def _():
            pltpu.async_copy(r_hbm.at[base], row_v.at[0], sem_r.at[0])

        def fire_write(row_i):
            q = lax.rem(row_i, 4)
            pltpu.async_copy(rows_v.at[q, pl.ds(0, K)],
                             nf_out.at[base + row_i], sem_w)

        def wait_gather(row_i):
            q = lax.rem(row_i, 4)
            pltpu.make_async_copy(x_hbm.at[nbr_v.at[q]], rows_v.at[q],
                                  sem_g).wait()

        def wait_write(row_i):
            q = lax.rem(row_i, 4)
            pltpu.make_async_copy(rows_v.at[q, pl.ds(0, K)],
                                  nf_out.at[base + row_i], sem_w).wait()

        def row_body(r, _):
            cur = lax.rem(r, 2)
            nxt = 1 - cur
            q = lax.rem(r, 4)
            pltpu.make_async_copy(r_hbm.at[base + r], row_v.at[cur],
                                  sem_r.at[cur]).wait()

            @pl.when(r + 1 < nrows)
            def _():
                pltpu.async_copy(r_hbm.at[base + r + 1], row_v.at[nxt],
                                 sem_r.at[nxt])

            def grp_body(g, lidx):
                off = g * (GRP * 16)
                vs = [row_v[cur, pl.ds(off + t * 16, 16)]
                      for t in range(GRP)]
                tot = vs[0]
                for t in range(1, GRP):
                    tot = tot + vs[t]
                nz = jnp.max(tot, axis=0) > 0

                @pl.when(nz)
                def _():
                    for t in range(GRP):
                        v = vs[t]
                        okm = (v > 0) & (v <= K)
                        pos = jnp.where(okm, v - 1, K + lanes)
                        plsc.store_scatter(nbr_v.at[q, pl.ds(0, 64)],
                                           [pos], lidx + t * 16)

                return lidx + GRP * 16

            lax.fori_loop(0, NUM_GRPS, grp_body, lanes)

            @pl.when(r >= 4)
            def _():
                wait_write(r - 4)

            pltpu.async_copy(x_hbm.at[nbr_v.at[q]], rows_v.at[q], sem_g)

            @pl.when(r >= 2)
            def _():
                wait_gather(r - 2)
                fire_write(r - 2)

            return 0

        lax.fori_loop(0, nrows, row_body, 0)

        wait_gather(nrows - 2)
        fire_write(nrows - 2)
        wait_gather(nrows - 1)
        fire_write(nrows - 1)
        for t in range(4):
            wait_write(nrows - 4 + t)

    return sc_kernel(r32, x)


def _tc_body(deg_ref, nf_ref, x_ref, W_ref, b_ref, out_ref, v_ref):
    deg = deg_ref[...]
    nf = nf_ref[...]
    xb = x_ref[...]
    pos = lax.broadcasted_iota(jnp.int32, (BLK, K), 1)
    degc = jnp.minimum(deg, K)
    valid = pos < degc
    m = jnp.maximum(jnp.where(deg > 5, (deg + 1) // 2, 1), 1)
    removed = ((pos + 1) % m) == 0
    keep = valid & jnp.logical_not(removed)
    nk = jnp.sum(keep.astype(jnp.int32), axis=1, keepdims=True)
    big = jnp.float32(1e30)
    keepf = keep.astype(jnp.float32)[:, :, None]
    v = nf * keepf + (1.0 - keepf) * big
    v_ref[...] = v
    maxd = jnp.max(degc)

    def cnt_body(k, carry):
        lt, le = carry
        vk = v_ref[:, pl.ds(k, 1), :]
        lt = lt + (vk < v).astype(jnp.float32)
        le = le + (vk <= v).astype(jnp.float32)
        return lt, le

    z = jnp.zeros((BLK, K, D), jnp.float32)
    lt, le = lax.fori_loop(0, maxd, cnt_body, (z, z))

    agg = jnp.zeros((BLK, D), jnp.float32)
    nkq = [(nk + 3) // 4 - 1, (nk + 1) // 2 - 1, (3 * nk + 3) // 4 - 1]
    for iq in nkq:
        iqf = jnp.where(nk > 0, iq, -1).astype(jnp.float32)[:, :, None]
        ind = ((lt <= iqf) & (iqf < le)).astype(jnp.float32)
        num = jnp.sum(v * ind, axis=1)
        den = jnp.sum(ind, axis=1)
        agg = agg + num / jnp.maximum(den, 1.0)
    agg = agg * jnp.float32(1.0 / 3.0)

    buf = 0.5 * agg + 0.5 * xb
    out = lax.dot_general(buf, W_ref[...], (((1,), (1,)), ((), ())),
                          preferred_element_type=jnp.float32)
    out_ref[...] = out + b_ref[...] + xb


def _tc_quantile_linear(nf, deg2d, x_pad, W, b_tile):
    grid = (NP // BLK,)
    return pl.pallas_call(
        _tc_body,
        grid=grid,
        in_specs=[
            pl.BlockSpec((BLK, 1), lambda i: (i, 0)),
            pl.BlockSpec((BLK, K, D), lambda i: (i, 0, 0)),
            pl.BlockSpec((BLK, D), lambda i: (i, 0)),
            pl.BlockSpec((D, D), lambda i: (0, 0)),
            pl.BlockSpec((BLK, D), lambda i: (0, 0)),
        ],
        out_specs=pl.BlockSpec((BLK, D), lambda i: (i, 0)),
        out_shape=jax.ShapeDtypeStruct((NP, D), jnp.float32),
        scratch_shapes=[pltpu.VMEM((BLK, K, D), jnp.float32)],
    )(deg2d, nf, x_pad, W, b_tile)


def kernel(x, A, W, b):
    r32, deg = _tc_rank(A)
    nf = _sc_extract_gather(r32, x)
    deg2d = jnp.pad(deg, ((0, NP - N), (0, 0)))
    x_pad = jnp.pad(x, ((0, NP - N), (0, 0)))
    b_tile = jnp.tile(b[None, :], (BLK, 1))
    out = _tc_quantile_linear(nf, deg2d, x_pad, W, b_tile)
    return out[:N]


# speedup vs baseline: 1.0969x; 1.0969x over previous
import functools

import jax
import jax.numpy as jnp
from jax import lax
from jax.experimental import pallas as pl
from jax.experimental.pallas import tpu as pltpu
from jax.experimental.pallas import tpu_sc as plsc

N = 10000
D = 128
K = 48
NP = 10240
ROWS_PER_W = 320
NUM_VREGS = 625
GRP = 5
NUM_GRPS = 125
BLK = 8


RBLK = 80


def _rank_body(A_ref, r_ref, deg_ref):
    a = A_ref[...] != 0.0
    inc = jnp.where(a, 1, 0)
    s = 1
    while s < N:
        sh = jnp.concatenate(
            [jnp.zeros((RBLK, s), jnp.int32), inc[:, :N - s]], axis=1)
        inc = inc + sh
        s *= 2
    r_ref[...] = jnp.where(a, inc, 0)
    deg_ref[...] = inc[:, N - 1:N]


def _tc_rank(A):
    return pl.pallas_call(
        _rank_body,
        grid=(N // RBLK,),
        in_specs=[pl.BlockSpec((RBLK, N), lambda i: (i, 0))],
        out_specs=[
            pl.BlockSpec((RBLK, N), lambda i: (i, 0)),
            pl.BlockSpec((RBLK, 1), lambda i: (i, 0)),
        ],
        out_shape=[
            jax.ShapeDtypeStruct((N, N), jnp.int32),
            jax.ShapeDtypeStruct((N, 1), jnp.int32),
        ],
    )(A)


def _sc_extract_gather(r32, x):
    info = plsc.get_sparse_core_info()
    nc = info.num_cores
    mesh = plsc.VectorSubcoreMesh(core_axis_name="c", subcore_axis_name="s")

    @functools.partial(
        pl.kernel,
        mesh=mesh,
        compiler_params=pltpu.CompilerParams(needs_layout_passes=False),
        out_type=jax.ShapeDtypeStruct((NP, K, D), jnp.float32),
        scratch_types=[
            pltpu.VMEM((N,), jnp.int32),
            pltpu.VMEM((64,), jnp.int32),
            pltpu.VMEM((64, D), jnp.float32),
            pltpu.SemaphoreType.DMA,
        ],
    )
    def sc_kernel(r_hbm, x_hbm, nf_out, row_v, nbr_v, rows_v, sem):
        wid = lax.axis_index("s") * nc + lax.axis_index("c")
        base = wid * ROWS_PER_W
        lanes = lax.iota(jnp.int32, 16)
        zeros16 = jnp.zeros((16,), jnp.int32)
        for t in range(4):
            nbr_v[pl.ds(t * 16, 16)] = zeros16

        def row_body(r, _):
            row = base + r

            @pl.when(row < N)
            def _():
                pltpu.sync_copy(r_hbm.at[row], row_v)

                def vreg_body(t, lane_idx):
                    v = row_v[pl.ds(t * 16, 16)]
                    okm = (v > 0) & (v <= K)
                    pos = jnp.where(okm, v - 1, K + lanes)
                    plsc.store_scatter(nbr_v.at[pl.ds(0, 64)], [pos],
                                       lane_idx)
                    return lane_idx + 16

                lax.fori_loop(0, NUM_VREGS, vreg_body, lanes)
                pltpu.async_copy(x_hbm.at[nbr_v], rows_v, sem).wait()
                pltpu.sync_copy(rows_v.at[pl.ds(0, K)], nf_out.at[row])

            return 0

        lax.fori_loop(0, ROWS_PER_W, row_body, 0)

    return sc_kernel(r32, x)


def _tc_body(deg_ref, nf_ref, x_ref, W_ref, b_ref, out_ref, v_ref):
    deg = deg_ref[...]
    nf = nf_ref[...]
    xb = x_ref[...]
    pos = lax.broadcasted_iota(jnp.int32, (BLK, K), 1)
    degc = jnp.minimum(deg, K)
    valid = pos < degc
    m = jnp.maximum(jnp.where(deg > 5, (deg + 1) // 2, 1), 1)
    removed = ((pos + 1) % m) == 0
    keep = valid & jnp.logical_not(removed)
    nk = jnp.sum(keep.astype(jnp.int32), axis=1, keepdims=True)
    big = jnp.float32(1e30)
    keepf = keep.astype(jnp.float32)[:, :, None]
    v = nf * keepf + (1.0 - keepf) * big
    v_ref[...] = v
    maxd = jnp.max(degc)

    def cnt_body(k, carry):
        lt, le = carry
        vk = v_ref[:, pl.ds(k, 1), :]
        lt = lt + (vk < v).astype(jnp.float32)
        le = le + (vk <= v).astype(jnp.float32)
        return lt, le

    z = jnp.zeros((BLK, K, D), jnp.float32)
    lt, le = lax.fori_loop(0, maxd, cnt_body, (z, z))

    agg = jnp.zeros((BLK, D), jnp.float32)
    nkq = [(nk + 3) // 4 - 1, (nk + 1) // 2 - 1, (3 * nk + 3) // 4 - 1]
    for iq in nkq:
        iqf = jnp.where(nk > 0, iq, -1).astype(jnp.float32)[:, :, None]
        ind = ((lt <= iqf) & (iqf < le)).astype(jnp.float32)
        num = jnp.sum(v * ind, axis=1)
        den = jnp.sum(ind, axis=1)
        agg = agg + num / jnp.maximum(den, 1.0)
    agg = agg * jnp.float32(1.0 / 3.0)

    buf = 0.5 * agg + 0.5 * xb
    out = lax.dot_general(buf, W_ref[...], (((1,), (1,)), ((), ())),
                          preferred_element_type=jnp.float32)
    out_ref[...] = out + b_ref[...] + xb


def _tc_quantile_linear(nf, deg2d, x_pad, W, b_tile):
    grid = (NP // BLK,)
    return pl.pallas_call(
        _tc_body,
        grid=grid,
        in_specs=[
            pl.BlockSpec((BLK, 1), lambda i: (i, 0)),
            pl.BlockSpec((BLK, K, D), lambda i: (i, 0, 0)),
            pl.BlockSpec((BLK, D), lambda i: (i, 0)),
            pl.BlockSpec((D, D), lambda i: (0, 0)),
            pl.BlockSpec((BLK, D), lambda i: (0, 0)),
        ],
        out_specs=pl.BlockSpec((BLK, D), lambda i: (i, 0)),
        out_shape=jax.ShapeDtypeStruct((NP, D), jnp.float32),
        scratch_shapes=[pltpu.VMEM((BLK, K, D), jnp.float32)],
    )(deg2d, nf, x_pad, W, b_tile)


def kernel(x, A, W, b):
    r32, deg = _tc_rank(A)
    nf = _sc_extract_gather(r32, x)
    deg2d = jnp.pad(deg, ((0, NP - N), (0, 0)))
    x_pad = jnp.pad(x, ((0, NP - N), (0, 0)))
    b_tile = jnp.tile(b[None, :], (BLK, 1))
    out = _tc_quantile_linear(nf, deg2d, x_pad, W, b_tile)
    return out[:N]
